# baseline (speedup 1.0000x reference)
import jax
import jax.numpy as jnp
from jax.experimental import pallas as pl
from jax.experimental.pallas import tpu as pltpu

_SIZES = [16, 16, 32, 64] + [128] * 12 + [64, 32, 16, 16]
_CHUNKS = []
_off = 0
for _s in _SIZES:
    _CHUNKS.append((_off, _s))
    _off += _s
_MAXC = max(_SIZES)


def _pipeline_kernel(x_hbm, e_ref, o_hbm, xb0, xb1, ob0, ob1, si0, si1, so0, so1):
    e = e_ref[...]
    xbufs, obufs = (xb0, xb1), (ob0, ob1)
    isems, osems = (si0, si1), (so0, so1)
    n = len(_CHUNKS)

    def in_copy(i):
        off, sz = _CHUNKS[i]
        return pltpu.make_async_copy(
            x_hbm.at[pl.ds(off, sz)], xbufs[i % 2].at[pl.ds(0, sz)], isems[i % 2]
        )

    def out_copy(i):
        off, sz = _CHUNKS[i]
        return pltpu.make_async_copy(
            obufs[i % 2].at[pl.ds(0, sz)], o_hbm.at[pl.ds(off, sz)], osems[i % 2]
        )

    in_copy(0).start()
    for i, (_, sz) in enumerate(_CHUNKS):
        if i + 1 < n:
            in_copy(i + 1).start()
        in_copy(i).wait()
        if i >= 2:
            out_copy(i - 2).wait()
        xb, ob = xbufs[i % 2], obufs[i % 2]

        def compute_plane(p, carry, xb=xb, ob=ob):
            ob[p] = jax.lax.dot_general(
                e,
                xb[p].astype(jnp.bfloat16),
                dimension_numbers=(((1,), (0,)), ((), ())),
                preferred_element_type=jnp.float32,
            )
            return carry

        jax.lax.fori_loop(0, sz, compute_plane, 0)
        out_copy(i).start()
    out_copy(n - 2).wait()
    out_copy(n - 1).wait()


def kernel(inputs, channel_embeddings):
    B, H, W, C = inputs.shape
    D = channel_embeddings.shape[1]
    P = B * H

    x_t = jnp.transpose(inputs, (0, 1, 3, 2)).reshape(P, C, W)
    e_t = jnp.transpose(channel_embeddings, (1, 0)).astype(jnp.bfloat16)

    out_t = pl.pallas_call(
        _pipeline_kernel,
        in_specs=[
            pl.BlockSpec(memory_space=pl.ANY),
            pl.BlockSpec(memory_space=pltpu.VMEM),
        ],
        out_specs=pl.BlockSpec(memory_space=pl.ANY),
        out_shape=jax.ShapeDtypeStruct((P, D, W), jnp.float32),
        scratch_shapes=[
            pltpu.VMEM((_MAXC, C, W), jnp.float32),
            pltpu.VMEM((_MAXC, C, W), jnp.float32),
            pltpu.VMEM((_MAXC, D, W), jnp.float32),
            pltpu.VMEM((_MAXC, D, W), jnp.float32),
            pltpu.SemaphoreType.DMA,
            pltpu.SemaphoreType.DMA,
            pltpu.SemaphoreType.DMA,
            pltpu.SemaphoreType.DMA,
        ],
    )(x_t, e_t)
    return jnp.transpose(out_t.reshape(B, H, D, W), (0, 1, 3, 2))


# speedup vs baseline: 3.7029x; 3.7029x over previous
import jax
import jax.numpy as jnp
from jax.experimental import pallas as pl
from jax.experimental.pallas import tpu as pltpu

_BLOCK_P = 128


def _contract_kernel(x_ref, e_ref, o_ref):
    e = e_ref[...]
    for p in range(x_ref.shape[0]):
        x = x_ref[p].astype(jnp.bfloat16)
        o_ref[p] = jax.lax.dot_general(
            e,
            x,
            dimension_numbers=(((1,), (0,)), ((), ())),
            preferred_element_type=jnp.float32,
        )


def kernel(inputs, channel_embeddings):
    B, H, W, C = inputs.shape
    D = channel_embeddings.shape[1]
    P = B * H

    x_t = jnp.transpose(inputs, (0, 1, 3, 2)).reshape(P, C, W)
    e_t = jnp.transpose(channel_embeddings, (1, 0)).astype(jnp.bfloat16)

    out_t = pl.pallas_call(
        _contract_kernel,
        grid=(P // _BLOCK_P,),
        in_specs=[
            pl.BlockSpec((_BLOCK_P, C, W), lambda i: (i, 0, 0)),
            pl.BlockSpec((D, C), lambda i: (0, 0)),
        ],
        out_specs=pl.BlockSpec((_BLOCK_P, D, W), lambda i: (i, 0, 0)),
        out_shape=jax.ShapeDtypeStruct((P, D, W), jnp.float32),
        compiler_params=pltpu.CompilerParams(
            dimension_semantics=("arbitrary",),
        ),
    )(x_t, e_t)
    return jnp.transpose(out_t.reshape(B, H, D, W), (0, 1, 3, 2))
